# baseline (speedup 1.0000x reference)
import functools

import jax
import jax.numpy as jnp
from jax import lax
from jax.experimental import pallas as pl
from jax.experimental.pallas import tpu as pltpu
from jax.experimental.pallas import tpu_sc as plsc

_C = 8
_R = 128
_V = _R * _R * _R
_NC = 2
_NS = 16
_NW = _NC * _NS
_L = 16
_B = 128


def _interp_body(xs, ys, zs, table, out, xv, yv, zv, fxv, fyv, fzv,
                 idxv, rowsv, outv, gsem):
  n_points = xs.shape[0]
  pts_per_w = n_points // _NW
  n_chunks = pts_per_w // _B
  wid = lax.axis_index("s") * _NC + lax.axis_index("c")

  def chunk_body(t, _):
    base = wid * pts_per_w + t * _B
    pltpu.sync_copy(xs.at[pl.ds(base, _B)], xv)
    pltpu.sync_copy(ys.at[pl.ds(base, _B)], yv)
    pltpu.sync_copy(zs.at[pl.ds(base, _B)], zv)

    def index_body(j, _):
      sl = pl.ds(j * _L, _L)
      gx = xv[sl]
      gy = yv[sl]
      gz = zv[sl]
      sx = (gx + 1.0) * 0.5 * (_R - 1)
      sy = (gy + 1.0) * 0.5 * (_R - 1)
      sz = (gz + 1.0) * 0.5 * (_R - 1)
      ix = jnp.minimum(sx.astype(jnp.int32), _R - 2)
      iy = jnp.minimum(sy.astype(jnp.int32), _R - 2)
      iz = jnp.minimum(sz.astype(jnp.int32), _R - 2)
      fxv[sl] = sx - ix.astype(jnp.float32)
      fyv[sl] = sy - iy.astype(jnp.float32)
      fzv[sl] = sz - iz.astype(jnp.float32)
      cell = (iz << 14) + (iy << 7) + ix
      prow = (cell >> 1) + ((cell & 1) << 20)
      idxv[0, sl] = prow
      idxv[1, sl] = prow + 64
      idxv[2, sl] = prow + 8192
      idxv[3, sl] = prow + 8192 + 64
      return 0

    lax.fori_loop(0, _B // _L, index_body, 0)

    copies = [
        pltpu.make_async_copy(table.at[idxv.at[k]], rowsv.at[k], gsem)
        for k in range(4)
    ]
    for cp in copies:
      cp.start()
    for cp in copies:
      cp.wait()

    lane = lax.iota(jnp.int32, _L)
    lo_half = lane < _C
    swap = lane ^ _C

    def combine_body(jj, _):
      sl = pl.ds(jj * _L, _L)
      fxr = fxv[sl]
      fyr = fyv[sl]
      fzr = fzv[sl]
      res_even = [None]
      for u in range(_L):
        splat = lane * 0 + u
        ex = fxr[splat]
        ey = fyr[splat]
        ez = fzr[splat]
        wx = jnp.where(lo_half, 1.0 - ex, ex)
        eyc = 1.0 - ey
        ezc = 1.0 - ez
        w00 = eyc * ezc
        w01 = ey * ezc
        w10 = eyc * ez
        w11 = ey * ez
        p = jj * _L + u
        r0 = rowsv[0, p, :]
        r1 = rowsv[1, p, :]
        r2 = rowsv[2, p, :]
        r3 = rowsv[3, p, :]
        tsum = w00 * r0 + w01 * r1 + w10 * r2 + w11 * r3
        acc = wx * tsum
        res = acc + acc[swap]
        if u % 2 == 0:
          res_even[0] = res
        else:
          merged = jnp.where(lo_half, res_even[0], res)
          outv[pl.ds((p - 1) * _C, _L)] = merged
      return 0

    lax.fori_loop(0, _B // _L, combine_body, 0)
    pltpu.sync_copy(outv, out.at[pl.ds(base * _C, _B * _C)])
    return 0

  lax.fori_loop(0, n_chunks, chunk_body, 0)


@functools.lru_cache(maxsize=None)
def _build(n_points):
  assert n_points % (_NW * _B) == 0
  mesh = plsc.VectorSubcoreMesh(
      core_axis_name="c", subcore_axis_name="s",
      num_cores=_NC, num_subcores=_NS)
  return pl.kernel(
      _interp_body,
      out_type=jax.ShapeDtypeStruct((n_points * _C,), jnp.float32),
      mesh=mesh,
      compiler_params=pltpu.CompilerParams(use_tc_tiling_on_sc=False),
      scratch_types=[
          pltpu.VMEM((_B,), jnp.float32),
          pltpu.VMEM((_B,), jnp.float32),
          pltpu.VMEM((_B,), jnp.float32),
          pltpu.VMEM((_B,), jnp.float32),
          pltpu.VMEM((_B,), jnp.float32),
          pltpu.VMEM((_B,), jnp.float32),
          pltpu.VMEM((4, _B), jnp.int32),
          pltpu.VMEM((4, _B, 2 * _C), jnp.float32),
          pltpu.VMEM((_B * _C,), jnp.float32),
          pltpu.SemaphoreType.DMA,
      ],
  )


def kernel(x, grid, bound):
  n = x.shape[0]
  xn = (x.astype(jnp.float32) / bound).T
  flat = jnp.transpose(grid[0], (1, 2, 3, 0)).reshape(-1)
  table = jnp.concatenate(
      [flat, flat[_C:], jnp.zeros((_C,), jnp.float32)]).reshape(_V, 2 * _C)
  out = _build(n)(xn[0], xn[1], xn[2], table)
  return out.reshape(x.shape[:-1] + (_C,))


# speedup vs baseline: 1.2425x; 1.2425x over previous
import functools

import jax
import jax.numpy as jnp
from jax import lax
from jax.experimental import pallas as pl
from jax.experimental.pallas import tpu as pltpu
from jax.experimental.pallas import tpu_sc as plsc

_C = 8
_R = 128
_V = _R * _R * _R
_NC = 2
_NS = 16
_NW = _NC * _NS
_L = 16
_B = 128


def _interp_body(xs, ys, zs, table, out, cv, fv, idxv, rowsv, outv,
                 csem0, csem1, gsem0, gsem1, osem0, osem1):
  n_points = xs.shape[0]
  pts_per_w = n_points // _NW
  n_chunks = pts_per_w // _B
  n_pairs = n_chunks // 2
  wid = lax.axis_index("s") * _NC + lax.axis_index("c")
  w_base = wid * pts_per_w

  csems = (csem0, csem1)
  gsems = (gsem0, gsem1)
  osems = (osem0, osem1)

  def coord_copies(t, par):
    base = w_base + t * _B
    sem = csems[par]
    return [
        pltpu.make_async_copy(xs.at[pl.ds(base, _B)], cv.at[par, 0], sem),
        pltpu.make_async_copy(ys.at[pl.ds(base, _B)], cv.at[par, 1], sem),
        pltpu.make_async_copy(zs.at[pl.ds(base, _B)], cv.at[par, 2], sem),
    ]

  def gather_copies(par):
    sem = gsems[par]
    return [
        pltpu.make_async_copy(table.at[idxv.at[par, k]], rowsv.at[par, k], sem)
        for k in range(4)
    ]

  def out_copy(t, par):
    base = w_base + t * _B
    return pltpu.make_async_copy(
        outv.at[par], out.at[pl.ds(base * _C, _B * _C)], osems[par])

  def fire(copies):
    for c in copies:
      c.start()

  def drain(copies):
    for c in copies:
      c.wait()

  def index_phase(par):
    def index_body(j, _):
      sl = pl.ds(j * _L, _L)
      gx = cv[par, 0, sl]
      gy = cv[par, 1, sl]
      gz = cv[par, 2, sl]
      sx = (gx + 1.0) * 0.5 * (_R - 1)
      sy = (gy + 1.0) * 0.5 * (_R - 1)
      sz = (gz + 1.0) * 0.5 * (_R - 1)
      ix = jnp.minimum(sx.astype(jnp.int32), _R - 2)
      iy = jnp.minimum(sy.astype(jnp.int32), _R - 2)
      iz = jnp.minimum(sz.astype(jnp.int32), _R - 2)
      fv[par, 0, sl] = sx - ix.astype(jnp.float32)
      fv[par, 1, sl] = sy - iy.astype(jnp.float32)
      fv[par, 2, sl] = sz - iz.astype(jnp.float32)
      cell = (iz << 14) + (iy << 7) + ix
      prow = (cell >> 1) + ((cell & 1) << 20)
      idxv[par, 0, sl] = prow
      idxv[par, 1, sl] = prow + 64
      idxv[par, 2, sl] = prow + 8192
      idxv[par, 3, sl] = prow + 8192 + 64
      return 0

    lax.fori_loop(0, _B // _L, index_body, 0)

  lane = lax.iota(jnp.int32, _L)
  lo_half = lane < _C
  swap = lane ^ _C

  def combine_phase(par):
    def combine_body(jj, _):
      sl = pl.ds(jj * _L, _L)
      fxr = fv[par, 0, sl]
      fyr = fv[par, 1, sl]
      fzr = fv[par, 2, sl]
      res_even = [None]
      for u in range(_L):
        splat = lane * 0 + u
        ex = fxr[splat]
        ey = fyr[splat]
        ez = fzr[splat]
        wx = jnp.where(lo_half, 1.0 - ex, ex)
        eyc = 1.0 - ey
        ezc = 1.0 - ez
        w00 = eyc * ezc
        w01 = ey * ezc
        w10 = eyc * ez
        w11 = ey * ez
        p = jj * _L + u
        r0 = rowsv[par, 0, p, :]
        r1 = rowsv[par, 1, p, :]
        r2 = rowsv[par, 2, p, :]
        r3 = rowsv[par, 3, p, :]
        tsum = w00 * r0 + w01 * r1 + w10 * r2 + w11 * r3
        acc = wx * tsum
        res = acc + acc[swap]
        if u % 2 == 0:
          res_even[0] = res
        else:
          merged = jnp.where(lo_half, res_even[0], res)
          outv[par, pl.ds((p - 1) * _C, _L)] = merged
      return 0

    lax.fori_loop(0, _B // _L, combine_body, 0)

  fire(coord_copies(0, 0))
  fire(coord_copies(1, 1))

  def pair_body(m, _):
    a = 2 * m
    b = a + 1

    drain(coord_copies(a, 0))
    index_phase(0)
    fire(gather_copies(0))

    @pl.when(m + 1 < n_pairs)
    def _():
      fire(coord_copies(a + 2, 0))

    @pl.when(m > 0)
    def _():
      drain(gather_copies(1))

      @pl.when(m > 1)
      def _():
        drain([out_copy(b - 4, 1)])

      combine_phase(1)
      fire([out_copy(b - 2, 1)])

    drain(coord_copies(b, 1))
    index_phase(1)
    fire(gather_copies(1))

    @pl.when(m + 1 < n_pairs)
    def _():
      fire(coord_copies(b + 2, 1))

    drain(gather_copies(0))

    @pl.when(m > 0)
    def _():
      drain([out_copy(a - 2, 0)])

    combine_phase(0)
    fire([out_copy(a, 0)])
    return 0

  lax.fori_loop(0, n_pairs, pair_body, 0)

  last = n_chunks - 1
  drain(gather_copies(1))
  drain([out_copy(last - 2, 1)])
  combine_phase(1)
  fire([out_copy(last, 1)])
  drain([out_copy(last - 1, 0)])
  drain([out_copy(last, 1)])


@functools.lru_cache(maxsize=None)
def _build(n_points):
  assert n_points % (_NW * _B * 2) == 0
  mesh = plsc.VectorSubcoreMesh(
      core_axis_name="c", subcore_axis_name="s",
      num_cores=_NC, num_subcores=_NS)
  return pl.kernel(
      _interp_body,
      out_type=jax.ShapeDtypeStruct((n_points * _C,), jnp.float32),
      mesh=mesh,
      compiler_params=pltpu.CompilerParams(use_tc_tiling_on_sc=False),
      scratch_types=[
          pltpu.VMEM((2, 3, _B), jnp.float32),
          pltpu.VMEM((2, 3, _B), jnp.float32),
          pltpu.VMEM((2, 4, _B), jnp.int32),
          pltpu.VMEM((2, 4, _B, 2 * _C), jnp.float32),
          pltpu.VMEM((2, _B * _C), jnp.float32),
          pltpu.SemaphoreType.DMA,
          pltpu.SemaphoreType.DMA,
          pltpu.SemaphoreType.DMA,
          pltpu.SemaphoreType.DMA,
          pltpu.SemaphoreType.DMA,
          pltpu.SemaphoreType.DMA,
      ],
  )


def kernel(x, grid, bound):
  n = x.shape[0]
  xn = (x.astype(jnp.float32) / bound).T
  flat = jnp.transpose(grid[0], (1, 2, 3, 0)).reshape(-1)
  table = jnp.concatenate(
      [flat, flat[_C:], jnp.zeros((_C,), jnp.float32)]).reshape(_V, 2 * _C)
  out = _build(n)(xn[0], xn[1], xn[2], table)
  return out.reshape(x.shape[:-1] + (_C,))
